# baseline (device time: 177195 ns/iter reference)
import jax
import jax.numpy as jnp
from jax import lax
from jax.experimental import pallas as pl
from jax.experimental.pallas import tpu as pltpu

N_DEV = 16
B, Sq, Hq, Hkv, Dh, D = 4, 256, 8, 2, 128, 1024
ROWS = B * Sq
CH = ROWS // N_DEV
SW = D + 128
SCALE = 0.08838834764831843
HOPS = N_DEV - 1


def kernel(x, Wq, Wo, K_ext, V_ext):
    def body(x_ref, wq_ref, wo_ref, k_ref, v_ref, out_ref,
             S, R, nbuf, rs_send, rs_recv, ag_send, ag_recv):
        d = lax.axis_index("i")
        left = lax.rem(d + N_DEV - 1, N_DEV)
        right = lax.rem(d + 1, N_DEV)

        barrier = pltpu.get_barrier_semaphore()
        for nbr in (left, right):
            pl.semaphore_signal(barrier, inc=1, device_id=(nbr,),
                                device_id_type=pl.DeviceIdType.MESH)
        pl.semaphore_wait(barrier, 2)

        for b in range(B):
            qb = lax.dot_general(x_ref[b], wq_ref[...],
                                 (((1,), (0,)), ((), ())),
                                 preferred_element_type=jnp.float32)
            for g in range(Hkv):
                kg = k_ref[b, :, g, :]
                vg = v_ref[b, :, g, :]
                for hh in range(Hq // Hkv):
                    h = g * (Hq // Hkv) + hh
                    qh = qb[:, h * Dh:(h + 1) * Dh]
                    s = lax.dot_general(qh, kg, (((1,), (1,)), ((), ())),
                                        preferred_element_type=jnp.float32)
                    p = jnp.exp(s * SCALE)
                    lh = jnp.sum(p, axis=1, keepdims=True)
                    a = lax.dot_general(p, vg, (((1,), (0,)), ((), ())),
                                        preferred_element_type=jnp.float32)
                    S[b * Sq:(b + 1) * Sq, h * Dh:(h + 1) * Dh] = a
                    S[b * Sq:(b + 1) * Sq, D + h:D + h + 1] = lh

        for t in range(HOPS):
            cs = lax.rem(d - t + N_DEV, N_DEV)
            cr = lax.rem(d - t - 1 + N_DEV, N_DEV)
            rdma = pltpu.make_async_remote_copy(
                src_ref=S.at[pl.ds(cs * CH, CH)],
                dst_ref=R.at[t],
                send_sem=rs_send.at[t],
                recv_sem=rs_recv.at[t],
                device_id=(right,),
                device_id_type=pl.DeviceIdType.MESH,
            )
            rdma.start()
            rdma.wait()
            S[pl.ds(cr * CH, CH), :] = S[pl.ds(cr * CH, CH), :] + R[t]

        r0 = right * CH
        for h in range(Hq):
            nbuf[:, h * Dh:(h + 1) * Dh] = (
                S[pl.ds(r0, CH), h * Dh:(h + 1) * Dh]
                / S[pl.ds(r0, CH), D + h:D + h + 1])
        o = lax.dot_general(nbuf[...], wo_ref[...], (((1,), (0,)), ((), ())),
                            preferred_element_type=jnp.float32)
        out_ref[pl.ds(r0, CH), :] = o

        for t in range(HOPS):
            cs = lax.rem(d + 1 - t + N_DEV, N_DEV)
            rdma = pltpu.make_async_remote_copy(
                src_ref=out_ref.at[pl.ds(cs * CH, CH)],
                dst_ref=out_ref.at[pl.ds(cs * CH, CH)],
                send_sem=ag_send.at[t],
                recv_sem=ag_recv.at[t],
                device_id=(right,),
                device_id_type=pl.DeviceIdType.MESH,
            )
            rdma.start()
            rdma.wait()

    flat = pl.pallas_call(
        body,
        out_shape=jax.ShapeDtypeStruct((ROWS, D), jnp.float32),
        in_specs=[pl.BlockSpec(memory_space=pltpu.VMEM)] * 5,
        out_specs=pl.BlockSpec(memory_space=pltpu.VMEM),
        scratch_shapes=[
            pltpu.VMEM((ROWS, SW), jnp.float32),
            pltpu.VMEM((HOPS, CH, SW), jnp.float32),
            pltpu.VMEM((CH, D), jnp.float32),
            pltpu.SemaphoreType.DMA((HOPS,)),
            pltpu.SemaphoreType.DMA((HOPS,)),
            pltpu.SemaphoreType.DMA((HOPS,)),
            pltpu.SemaphoreType.DMA((HOPS,)),
        ],
        compiler_params=pltpu.CompilerParams(collective_id=0),
    )(x, Wq, Wo, K_ext, V_ext)
    return flat.reshape(B, Sq, D)


# device time: 132859 ns/iter; 1.3337x vs baseline; 1.3337x over previous
import jax
import jax.numpy as jnp
from jax import lax
from jax.experimental import pallas as pl
from jax.experimental.pallas import tpu as pltpu

N_DEV = 16
B, Sq, Hq, Hkv, Dh, D = 4, 256, 8, 2, 128, 1024
ROWS = B * Sq
CH = ROWS // N_DEV
SW = D + 128
WL = 640
WR = SW - WL
HA = D // 2
SCALE = 0.08838834764831843
HOPS = N_DEV - 1

RING = [0, 4, 8, 12, 15, 11, 7, 3, 2, 6, 10, 14, 13, 9, 5, 1]
POS = [0] * N_DEV
for _i, _dev in enumerate(RING):
    POS[_dev] = _i
RIGHT = [0] * N_DEV
LEFT = [0] * N_DEV
for _i, _dev in enumerate(RING):
    RIGHT[_dev] = RING[(_i + 1) % N_DEV]
    LEFT[_dev] = RING[(_i - 1) % N_DEV]


def kernel(x, Wq, Wo, K_ext, V_ext):
    def body(x_ref, wq_ref, wo_ref, k_ref, v_ref, out_ref,
             S, RL, RR, nbuf,
             rsa_s, rsa_r, rsb_s, rsb_r, aga_s, aga_r, agb_s, agb_r):
        d = lax.axis_index("i")

        def lut(idx, table):
            v = jnp.int32(table[-1])
            for k in range(len(table) - 2, -1, -1):
                v = jnp.where(idx == k, jnp.int32(table[k]), v)
            return v

        p = lut(d, POS)
        rd = lut(d, RIGHT)
        ld = lut(d, LEFT)

        barrier = pltpu.get_barrier_semaphore()
        for nbr in (ld, rd):
            pl.semaphore_signal(barrier, inc=1, device_id=(nbr,),
                                device_id_type=pl.DeviceIdType.MESH)
        pl.semaphore_wait(barrier, 2)

        for b in range(B):
            qb = lax.dot_general(x_ref[b], wq_ref[...],
                                 (((1,), (0,)), ((), ())),
                                 preferred_element_type=jnp.float32)
            for g in range(Hkv):
                kg = k_ref[b, :, g, :]
                vg = v_ref[b, :, g, :]
                for hh in range(Hq // Hkv):
                    h = g * (Hq // Hkv) + hh
                    qh = qb[:, h * Dh:(h + 1) * Dh]
                    s = lax.dot_general(qh, kg, (((1,), (1,)), ((), ())),
                                        preferred_element_type=jnp.float32)
                    p_ = jnp.exp(s * SCALE)
                    lh = jnp.sum(p_, axis=1, keepdims=True)
                    a = lax.dot_general(p_, vg, (((1,), (0,)), ((), ())),
                                        preferred_element_type=jnp.float32)
                    S[b * Sq:(b + 1) * Sq, h * Dh:(h + 1) * Dh] = a
                    S[b * Sq:(b + 1) * Sq, D + h:D + h + 1] = lh

        for t in range(HOPS):
            csr = lax.rem(p - t + N_DEV, N_DEV)
            csl = lax.rem(p + t + 2, N_DEV)
            rr = pltpu.make_async_remote_copy(
                src_ref=S.at[pl.ds(csr * CH, CH), pl.ds(0, WL)],
                dst_ref=RL.at[t],
                send_sem=rsa_s.at[t], recv_sem=rsa_r.at[t],
                device_id=(rd,), device_id_type=pl.DeviceIdType.MESH)
            rl = pltpu.make_async_remote_copy(
                src_ref=S.at[pl.ds(csl * CH, CH), pl.ds(WL, WR)],
                dst_ref=RR.at[t],
                send_sem=rsb_s.at[t], recv_sem=rsb_r.at[t],
                device_id=(ld,), device_id_type=pl.DeviceIdType.MESH)
            rr.start()
            rl.start()
            rr.wait()
            rl.wait()
            crr = lax.rem(p - t - 1 + N_DEV, N_DEV)
            crl = lax.rem(p + t + 3, N_DEV)
            S[pl.ds(crr * CH, CH), 0:WL] = (
                S[pl.ds(crr * CH, CH), 0:WL] + RL[t])
            S[pl.ds(crl * CH, CH), WL:SW] = (
                S[pl.ds(crl * CH, CH), WL:SW] + RR[t])

        r0 = lax.rem(p + 1, N_DEV) * CH
        for h in range(Hq):
            nbuf[:, h * Dh:(h + 1) * Dh] = (
                S[pl.ds(r0, CH), h * Dh:(h + 1) * Dh]
                / S[pl.ds(r0, CH), D + h:D + h + 1])
        o = lax.dot_general(nbuf[...], wo_ref[...], (((1,), (0,)), ((), ())),
                            preferred_element_type=jnp.float32)
        out_ref[pl.ds(r0, CH), :] = o

        for t in range(HOPS):
            csr = lax.rem(p + 1 - t + N_DEV, N_DEV)
            csl = lax.rem(p + 1 + t, N_DEV)
            ar = pltpu.make_async_remote_copy(
                src_ref=out_ref.at[pl.ds(csr * CH, CH), pl.ds(0, HA)],
                dst_ref=out_ref.at[pl.ds(csr * CH, CH), pl.ds(0, HA)],
                send_sem=aga_s.at[t], recv_sem=aga_r.at[t],
                device_id=(rd,), device_id_type=pl.DeviceIdType.MESH)
            al = pltpu.make_async_remote_copy(
                src_ref=out_ref.at[pl.ds(csl * CH, CH), pl.ds(HA, HA)],
                dst_ref=out_ref.at[pl.ds(csl * CH, CH), pl.ds(HA, HA)],
                send_sem=agb_s.at[t], recv_sem=agb_r.at[t],
                device_id=(ld,), device_id_type=pl.DeviceIdType.MESH)
            ar.start()
            al.start()
            ar.wait()
            al.wait()

    flat = pl.pallas_call(
        body,
        out_shape=jax.ShapeDtypeStruct((ROWS, D), jnp.float32),
        in_specs=[pl.BlockSpec(memory_space=pltpu.VMEM)] * 5,
        out_specs=pl.BlockSpec(memory_space=pltpu.VMEM),
        scratch_shapes=[
            pltpu.VMEM((ROWS, SW), jnp.float32),
            pltpu.VMEM((HOPS, CH, WL), jnp.float32),
            pltpu.VMEM((HOPS, CH, WR), jnp.float32),
            pltpu.VMEM((CH, D), jnp.float32),
            pltpu.SemaphoreType.DMA((HOPS,)),
            pltpu.SemaphoreType.DMA((HOPS,)),
            pltpu.SemaphoreType.DMA((HOPS,)),
            pltpu.SemaphoreType.DMA((HOPS,)),
            pltpu.SemaphoreType.DMA((HOPS,)),
            pltpu.SemaphoreType.DMA((HOPS,)),
            pltpu.SemaphoreType.DMA((HOPS,)),
            pltpu.SemaphoreType.DMA((HOPS,)),
        ],
        compiler_params=pltpu.CompilerParams(collective_id=0),
    )(x, Wq, Wo, K_ext, V_ext)
    return flat.reshape(B, Sq, D)


# device time: 109021 ns/iter; 1.6253x vs baseline; 1.2187x over previous
import jax
import jax.numpy as jnp
from jax import lax
from jax.experimental import pallas as pl
from jax.experimental.pallas import tpu as pltpu

N_DEV = 16
B, Sq, Hq, Hkv, Dh, D = 4, 256, 8, 2, 128, 1024
ROWS = B * Sq
CH = ROWS // N_DEV
SW = D + 128
SCALE = 0.08838834764831843

RING = [0, 4, 8, 12, 15, 11, 7, 3, 2, 6, 10, 14, 13, 9, 5, 1]
POS = [0] * N_DEV
for _i, _dev in enumerate(RING):
    POS[_dev] = _i
RIGHT = [0] * N_DEV
LEFT = [0] * N_DEV
for _i, _dev in enumerate(RING):
    RIGHT[_dev] = RING[(_i + 1) % N_DEV]
    LEFT[_dev] = RING[(_i - 1) % N_DEV]


def kernel(x, Wq, Wo, K_ext, V_ext):
    def body(x_ref, wq_ref, wo_ref, k_ref, v_ref, out_ref,
             S, LB, RB, nbuf,
             rsa_s, rsa_r, rsb_s, rsb_r, aga_s, aga_r, agb_s, agb_r):
        d = lax.axis_index("i")

        def lut(idx, table):
            v = jnp.int32(table[-1])
            for k in range(len(table) - 2, -1, -1):
                v = jnp.where(idx == k, jnp.int32(table[k]), v)
            return v

        p = lut(d, POS)
        rd = lut(d, RIGHT)
        ld = lut(d, LEFT)

        barrier = pltpu.get_barrier_semaphore()
        for nbr in (ld, rd):
            pl.semaphore_signal(barrier, inc=1, device_id=(nbr,),
                                device_id_type=pl.DeviceIdType.MESH)
        pl.semaphore_wait(barrier, 2)

        for b in range(B):
            qb = lax.dot_general(x_ref[b], wq_ref[...],
                                 (((1,), (0,)), ((), ())),
                                 preferred_element_type=jnp.float32)
            for g in range(Hkv):
                kg = k_ref[b, :, g, :]
                vg = v_ref[b, :, g, :]
                for hh in range(Hq // Hkv):
                    h = g * (Hq // Hkv) + hh
                    qh = qb[:, h * Dh:(h + 1) * Dh]
                    s = lax.dot_general(qh, kg, (((1,), (1,)), ((), ())),
                                        preferred_element_type=jnp.float32)
                    p_ = jnp.exp(s * SCALE)
                    lh = jnp.sum(p_, axis=1, keepdims=True)
                    a = lax.dot_general(p_, vg, (((1,), (0,)), ((), ())),
                                        preferred_element_type=jnp.float32)
                    S[b * Sq:(b + 1) * Sq, h * Dh:(h + 1) * Dh] = a
                    S[b * Sq:(b + 1) * Sq, D + h:D + h + 1] = lh

        for t in range(8):
            lw = pltpu.make_async_remote_copy(
                src_ref=S.at[pl.ds(lax.rem(p - 7 + t + N_DEV, N_DEV) * CH,
                                   CH)],
                dst_ref=LB.at[t],
                send_sem=rsa_s.at[t], recv_sem=rsa_r.at[t],
                device_id=(ld,), device_id_type=pl.DeviceIdType.MESH)
            lw.start()
            if t < 7:
                rw = pltpu.make_async_remote_copy(
                    src_ref=S.at[pl.ds(lax.rem(p + 8 - t, N_DEV) * CH, CH)],
                    dst_ref=RB.at[t],
                    send_sem=rsb_s.at[t], recv_sem=rsb_r.at[t],
                    device_id=(rd,), device_id_type=pl.DeviceIdType.MESH)
                rw.start()
            lw.wait()
            clw = lax.rem(p - 6 + t + N_DEV, N_DEV)
            S[pl.ds(clw * CH, CH), :] = S[pl.ds(clw * CH, CH), :] + LB[t]
            if t < 7:
                rw.wait()
                crw = lax.rem(p + 7 - t, N_DEV)
                S[pl.ds(crw * CH, CH), :] = (
                    S[pl.ds(crw * CH, CH), :] + RB[t])

        r0 = lax.rem(p + 1, N_DEV) * CH
        for h in range(Hq):
            nbuf[:, h * Dh:(h + 1) * Dh] = (
                S[pl.ds(r0, CH), h * Dh:(h + 1) * Dh]
                / S[pl.ds(r0, CH), D + h:D + h + 1])
        o = lax.dot_general(nbuf[...], wo_ref[...], (((1,), (0,)), ((), ())),
                            preferred_element_type=jnp.float32)
        out_ref[pl.ds(r0, CH), :] = o

        for t in range(8):
            csr = lax.rem(p + 1 - t + N_DEV, N_DEV)
            ar = pltpu.make_async_remote_copy(
                src_ref=out_ref.at[pl.ds(csr * CH, CH)],
                dst_ref=out_ref.at[pl.ds(csr * CH, CH)],
                send_sem=aga_s.at[t], recv_sem=aga_r.at[t],
                device_id=(rd,), device_id_type=pl.DeviceIdType.MESH)
            ar.start()
            if t < 7:
                csl = lax.rem(p + 1 + t, N_DEV)
                al = pltpu.make_async_remote_copy(
                    src_ref=out_ref.at[pl.ds(csl * CH, CH)],
                    dst_ref=out_ref.at[pl.ds(csl * CH, CH)],
                    send_sem=agb_s.at[t], recv_sem=agb_r.at[t],
                    device_id=(ld,), device_id_type=pl.DeviceIdType.MESH)
                al.start()
            ar.wait()
            if t < 7:
                al.wait()

    flat = pl.pallas_call(
        body,
        out_shape=jax.ShapeDtypeStruct((ROWS, D), jnp.float32),
        in_specs=[pl.BlockSpec(memory_space=pltpu.VMEM)] * 5,
        out_specs=pl.BlockSpec(memory_space=pltpu.VMEM),
        scratch_shapes=[
            pltpu.VMEM((ROWS, SW), jnp.float32),
            pltpu.VMEM((8, CH, SW), jnp.float32),
            pltpu.VMEM((7, CH, SW), jnp.float32),
            pltpu.VMEM((CH, D), jnp.float32),
            pltpu.SemaphoreType.DMA((8,)),
            pltpu.SemaphoreType.DMA((8,)),
            pltpu.SemaphoreType.DMA((7,)),
            pltpu.SemaphoreType.DMA((7,)),
            pltpu.SemaphoreType.DMA((8,)),
            pltpu.SemaphoreType.DMA((8,)),
            pltpu.SemaphoreType.DMA((7,)),
            pltpu.SemaphoreType.DMA((7,)),
        ],
        compiler_params=pltpu.CompilerParams(collective_id=0),
    )(x, Wq, Wo, K_ext, V_ext)
    return flat.reshape(B, Sq, D)


# device time: 88023 ns/iter; 2.0131x vs baseline; 1.2386x over previous
import jax
import jax.numpy as jnp
from jax import lax
from jax.experimental import pallas as pl
from jax.experimental.pallas import tpu as pltpu

N_DEV = 16
B, Sq, Hq, Hkv, Dh, D = 4, 256, 8, 2, 128, 1024
ROWS = B * Sq
CH = ROWS // N_DEV
SW = D + 128
SCALE = 0.08838834764831843

RING = [0, 4, 8, 12, 15, 11, 7, 3, 2, 6, 10, 14, 13, 9, 5, 1]
POS = [0] * N_DEV
for _i, _dev in enumerate(RING):
    POS[_dev] = _i
RIGHT = [0] * N_DEV
LEFT = [0] * N_DEV
for _i, _dev in enumerate(RING):
    RIGHT[_dev] = RING[(_i + 1) % N_DEV]
    LEFT[_dev] = RING[(_i - 1) % N_DEV]


def kernel(x, Wq, Wo, K_ext, V_ext):
    def body(x_ref, wq_ref, wo_ref, k_ref, v_ref, out_ref,
             S, LB, RB, nbuf,
             rsa_s, rsa_r, rsb_s, rsb_r, aga_s, aga_r, agb_s, agb_r):
        d = lax.axis_index("i")

        def lut(idx, table):
            v = jnp.int32(table[-1])
            for k in range(len(table) - 2, -1, -1):
                v = jnp.where(idx == k, jnp.int32(table[k]), v)
            return v

        p = lut(d, POS)
        rd = lut(d, RIGHT)
        ld = lut(d, LEFT)

        barrier = pltpu.get_barrier_semaphore()
        for nbr in (ld, rd):
            pl.semaphore_signal(barrier, inc=1, device_id=(nbr,),
                                device_id_type=pl.DeviceIdType.MESH)
        pl.semaphore_wait(barrier, 2)

        for b in range(B):
            qb = lax.dot_general(x_ref[b], wq_ref[...],
                                 (((1,), (0,)), ((), ())),
                                 preferred_element_type=jnp.float32)
            for g in range(Hkv):
                kg = k_ref[b, :, g, :]
                vg = v_ref[b, :, g, :]
                for hh in range(Hq // Hkv):
                    h = g * (Hq // Hkv) + hh
                    qh = qb[:, h * Dh:(h + 1) * Dh]
                    s = lax.dot_general(qh, kg, (((1,), (1,)), ((), ())),
                                        preferred_element_type=jnp.float32)
                    p_ = jnp.exp(s * SCALE)
                    lh = jnp.sum(p_, axis=1, keepdims=True)
                    a = lax.dot_general(p_, vg, (((1,), (0,)), ((), ())),
                                        preferred_element_type=jnp.float32)
                    S[b * Sq:(b + 1) * Sq, h * Dh:(h + 1) * Dh] = a
                    S[b * Sq:(b + 1) * Sq, D + h:D + h + 1] = lh

        HC = CH // 2

        def rs_copy(stream, t, half):
            if stream == "lw":
                cs = lax.rem(p - 7 + t + N_DEV, N_DEV)
                buf, ssem, rsem, dev = LB, rsa_s, rsa_r, ld
            else:
                cs = lax.rem(p + 8 - t, N_DEV)
                buf, ssem, rsem, dev = RB, rsb_s, rsb_r, rd
            return pltpu.make_async_remote_copy(
                src_ref=S.at[pl.ds(cs * CH + half * HC, HC)],
                dst_ref=buf.at[t, pl.ds(half * HC, HC)],
                send_sem=ssem.at[2 * t + half],
                recv_sem=rsem.at[2 * t + half],
                device_id=(dev,), device_id_type=pl.DeviceIdType.MESH)

        def rs_acc(stream, t, half):
            if stream == "lw":
                c = lax.rem(p - 6 + t + N_DEV, N_DEV)
                buf = LB
            else:
                c = lax.rem(p + 7 - t, N_DEV)
                buf = RB
            r = c * CH + half * HC
            S[pl.ds(r, HC), :] = (
                S[pl.ds(r, HC), :] + buf[t, half * HC:(half + 1) * HC, :])

        rs_d = {}
        for half in (0, 1):
            for stream in ("lw", "rw"):
                rs_d[(stream, 0, half)] = rs_copy(stream, 0, half)
                rs_d[(stream, 0, half)].start()
        for t in range(8):
            for half in (0, 1):
                for stream, steps in (("lw", 8), ("rw", 7)):
                    if t >= steps:
                        continue
                    rs_d[(stream, t, half)].wait()
                    rs_acc(stream, t, half)
                    if t + 1 < steps:
                        nxt = rs_copy(stream, t + 1, half)
                        rs_d[(stream, t + 1, half)] = nxt
                        nxt.start()

        r0 = lax.rem(p + 1, N_DEV) * CH
        for h in range(Hq):
            nbuf[:, h * Dh:(h + 1) * Dh] = (
                S[pl.ds(r0, CH), h * Dh:(h + 1) * Dh]
                / S[pl.ds(r0, CH), D + h:D + h + 1])
        o = lax.dot_general(nbuf[...], wo_ref[...], (((1,), (0,)), ((), ())),
                            preferred_element_type=jnp.float32)
        out_ref[pl.ds(r0, CH), :] = o

        def ag_copy(stream, t, half):
            if stream == "ar":
                cs = lax.rem(p + 1 - t + N_DEV, N_DEV)
                ssem, rsem, dev = aga_s, aga_r, rd
            else:
                cs = lax.rem(p + 1 + t, N_DEV)
                ssem, rsem, dev = agb_s, agb_r, ld
            sl = out_ref.at[pl.ds(cs * CH + half * HC, HC)]
            return pltpu.make_async_remote_copy(
                src_ref=sl, dst_ref=sl,
                send_sem=ssem.at[2 * t + half],
                recv_sem=rsem.at[2 * t + half],
                device_id=(dev,), device_id_type=pl.DeviceIdType.MESH)

        ag_d = {}
        for half in (0, 1):
            for stream in ("ar", "al"):
                ag_d[(stream, 0, half)] = ag_copy(stream, 0, half)
                ag_d[(stream, 0, half)].start()
        for t in range(8):
            for half in (0, 1):
                for stream, steps in (("ar", 8), ("al", 7)):
                    if t >= steps:
                        continue
                    ag_d[(stream, t, half)].wait()
                    if t + 1 < steps:
                        nxt = ag_copy(stream, t + 1, half)
                        ag_d[(stream, t + 1, half)] = nxt
                        nxt.start()

    flat = pl.pallas_call(
        body,
        out_shape=jax.ShapeDtypeStruct((ROWS, D), jnp.float32),
        in_specs=[pl.BlockSpec(memory_space=pltpu.VMEM)] * 5,
        out_specs=pl.BlockSpec(memory_space=pltpu.VMEM),
        scratch_shapes=[
            pltpu.VMEM((ROWS, SW), jnp.float32),
            pltpu.VMEM((8, CH, SW), jnp.float32),
            pltpu.VMEM((7, CH, SW), jnp.float32),
            pltpu.VMEM((CH, D), jnp.float32),
            pltpu.SemaphoreType.DMA((16,)),
            pltpu.SemaphoreType.DMA((16,)),
            pltpu.SemaphoreType.DMA((14,)),
            pltpu.SemaphoreType.DMA((14,)),
            pltpu.SemaphoreType.DMA((16,)),
            pltpu.SemaphoreType.DMA((16,)),
            pltpu.SemaphoreType.DMA((14,)),
            pltpu.SemaphoreType.DMA((14,)),
        ],
        compiler_params=pltpu.CompilerParams(collective_id=0),
    )(x, Wq, Wo, K_ext, V_ext)
    return flat.reshape(B, Sq, D)


# device time: 83163 ns/iter; 2.1307x vs baseline; 1.0584x over previous
import jax
import jax.numpy as jnp
from jax import lax
from jax.experimental import pallas as pl
from jax.experimental.pallas import tpu as pltpu

N_DEV = 16
B, Sq, Hq, Hkv, Dh, D = 4, 256, 8, 2, 128, 1024
ROWS = B * Sq
CH = ROWS // N_DEV
SW = D + 128
SCALE = 0.08838834764831843

RING = [0, 4, 8, 12, 15, 11, 7, 3, 2, 6, 10, 14, 13, 9, 5, 1]
POS = [0] * N_DEV
for _i, _dev in enumerate(RING):
    POS[_dev] = _i
RIGHT = [0] * N_DEV
LEFT = [0] * N_DEV
for _i, _dev in enumerate(RING):
    RIGHT[_dev] = RING[(_i + 1) % N_DEV]
    LEFT[_dev] = RING[(_i - 1) % N_DEV]


def kernel(x, Wq, Wo, K_ext, V_ext):
    def body(x_ref, wq_ref, wo_ref, k_ref, v_ref, out_ref,
             S, LB, RB, nbuf,
             rsa_s, rsa_r, rsb_s, rsb_r, aga_s, aga_r, agb_s, agb_r):
        d = lax.axis_index("i")

        def lut(idx, table):
            v = jnp.int32(table[-1])
            for k in range(len(table) - 2, -1, -1):
                v = jnp.where(idx == k, jnp.int32(table[k]), v)
            return v

        p = lut(d, POS)
        rd = lut(d, RIGHT)
        ld = lut(d, LEFT)

        barrier = pltpu.get_barrier_semaphore()
        for nbr in (ld, rd):
            pl.semaphore_signal(barrier, inc=1, device_id=(nbr,),
                                device_id_type=pl.DeviceIdType.MESH)
        pl.semaphore_wait(barrier, 2)

        def compute_batch(b):
            qb = lax.dot_general(x_ref[b], wq_ref[...],
                                 (((1,), (0,)), ((), ())),
                                 preferred_element_type=jnp.float32)
            for g in range(Hkv):
                kg = k_ref[b, :, g, :]
                vg = v_ref[b, :, g, :]
                for hh in range(Hq // Hkv):
                    h = g * (Hq // Hkv) + hh
                    qh = qb[:, h * Dh:(h + 1) * Dh]
                    s = lax.dot_general(qh, kg, (((1,), (1,)), ((), ())),
                                        preferred_element_type=jnp.float32)
                    p_ = jnp.exp(s * SCALE)
                    lh = jnp.sum(p_, axis=1, keepdims=True)
                    a = lax.dot_general(p_, vg, (((1,), (0,)), ((), ())),
                                        preferred_element_type=jnp.float32)
                    S[pl.ds(b * Sq, Sq), h * Dh:(h + 1) * Dh] = a
                    S[pl.ds(b * Sq, Sq), D + h:D + h + 1] = lh

        bp = lax.div(p, jnp.int32(B))
        compute_batch(lax.rem(bp + 2, jnp.int32(B)))
        compute_batch(lax.rem(bp + 3, jnp.int32(B)))

        HC = CH // 2

        def rs_copy(stream, t, half):
            if stream == "lw":
                cs = lax.rem(p - 7 + t + N_DEV, N_DEV)
                buf, ssem, rsem, dev = LB, rsa_s, rsa_r, ld
            else:
                cs = lax.rem(p + 8 - t, N_DEV)
                buf, ssem, rsem, dev = RB, rsb_s, rsb_r, rd
            return pltpu.make_async_remote_copy(
                src_ref=S.at[pl.ds(cs * CH + half * HC, HC)],
                dst_ref=buf.at[t, pl.ds(half * HC, HC)],
                send_sem=ssem.at[2 * t + half],
                recv_sem=rsem.at[2 * t + half],
                device_id=(dev,), device_id_type=pl.DeviceIdType.MESH)

        def rs_acc(stream, t, half):
            if stream == "lw":
                c = lax.rem(p - 6 + t + N_DEV, N_DEV)
                buf = LB
            else:
                c = lax.rem(p + 7 - t, N_DEV)
                buf = RB
            r = c * CH + half * HC
            S[pl.ds(r, HC), :] = (
                S[pl.ds(r, HC), :] + buf[t, half * HC:(half + 1) * HC, :])

        rs_d = {}

        def rs_step(t):
            for half in (0, 1):
                for stream, steps in (("lw", 8), ("rw", 7)):
                    if t >= steps:
                        continue
                    rs_d[(stream, t, half)].wait()
                    rs_acc(stream, t, half)
                    if t + 1 < steps:
                        nxt = rs_copy(stream, t + 1, half)
                        rs_d[(stream, t + 1, half)] = nxt
                        nxt.start()

        for half in (0, 1):
            for stream in ("lw", "rw"):
                rs_d[(stream, 0, half)] = rs_copy(stream, 0, half)
                rs_d[(stream, 0, half)].start()
        compute_batch(lax.rem(bp + 1, jnp.int32(B)))
        rs_step(0)
        compute_batch(bp)
        for t in range(1, 8):
            rs_step(t)

        r0 = lax.rem(p + 1, N_DEV) * CH
        for h in range(Hq):
            nbuf[:, h * Dh:(h + 1) * Dh] = (
                S[pl.ds(r0, CH), h * Dh:(h + 1) * Dh]
                / S[pl.ds(r0, CH), D + h:D + h + 1])
        o = lax.dot_general(nbuf[...], wo_ref[...], (((1,), (0,)), ((), ())),
                            preferred_element_type=jnp.float32)
        out_ref[pl.ds(r0, CH), :] = o

        def ag_copy(stream, t, half):
            if stream == "ar":
                cs = lax.rem(p + 1 - t + N_DEV, N_DEV)
                ssem, rsem, dev = aga_s, aga_r, rd
            else:
                cs = lax.rem(p + 1 + t, N_DEV)
                ssem, rsem, dev = agb_s, agb_r, ld
            sl = out_ref.at[pl.ds(cs * CH + half * HC, HC)]
            return pltpu.make_async_remote_copy(
                src_ref=sl, dst_ref=sl,
                send_sem=ssem.at[2 * t + half],
                recv_sem=rsem.at[2 * t + half],
                device_id=(dev,), device_id_type=pl.DeviceIdType.MESH)

        ag_d = {}
        for half in (0, 1):
            for stream in ("ar", "al"):
                ag_d[(stream, 0, half)] = ag_copy(stream, 0, half)
                ag_d[(stream, 0, half)].start()
        for t in range(8):
            for half in (0, 1):
                for stream, steps in (("ar", 8), ("al", 7)):
                    if t >= steps:
                        continue
                    ag_d[(stream, t, half)].wait()
                    if t + 1 < steps:
                        nxt = ag_copy(stream, t + 1, half)
                        ag_d[(stream, t + 1, half)] = nxt
                        nxt.start()

    flat = pl.pallas_call(
        body,
        out_shape=jax.ShapeDtypeStruct((ROWS, D), jnp.float32),
        in_specs=[pl.BlockSpec(memory_space=pltpu.VMEM)] * 5,
        out_specs=pl.BlockSpec(memory_space=pltpu.VMEM),
        scratch_shapes=[
            pltpu.VMEM((ROWS, SW), jnp.float32),
            pltpu.VMEM((8, CH, SW), jnp.float32),
            pltpu.VMEM((7, CH, SW), jnp.float32),
            pltpu.VMEM((CH, D), jnp.float32),
            pltpu.SemaphoreType.DMA((16,)),
            pltpu.SemaphoreType.DMA((16,)),
            pltpu.SemaphoreType.DMA((14,)),
            pltpu.SemaphoreType.DMA((14,)),
            pltpu.SemaphoreType.DMA((16,)),
            pltpu.SemaphoreType.DMA((16,)),
            pltpu.SemaphoreType.DMA((14,)),
            pltpu.SemaphoreType.DMA((14,)),
        ],
        compiler_params=pltpu.CompilerParams(collective_id=0),
    )(x, Wq, Wo, K_ext, V_ext)
    return flat.reshape(B, Sq, D)


# device time: 76973 ns/iter; 2.3020x vs baseline; 1.0804x over previous
import jax
import jax.numpy as jnp
from jax import lax
from jax.experimental import pallas as pl
from jax.experimental.pallas import tpu as pltpu

N_DEV = 16
B, Sq, Hq, Hkv, Dh, D = 4, 256, 8, 2, 128, 1024
ROWS = B * Sq
CH = ROWS // N_DEV
SW = D + 128
SCALE = 0.08838834764831843

RING = [0, 4, 8, 12, 15, 11, 7, 3, 2, 6, 10, 14, 13, 9, 5, 1]
POS = [0] * N_DEV
for _i, _dev in enumerate(RING):
    POS[_dev] = _i
RIGHT = [0] * N_DEV
LEFT = [0] * N_DEV
for _i, _dev in enumerate(RING):
    RIGHT[_dev] = RING[(_i + 1) % N_DEV]
    LEFT[_dev] = RING[(_i - 1) % N_DEV]


def kernel(x, Wq, Wo, K_ext, V_ext):
    def body(x_ref, wq_ref, wo_ref, k_ref, v_ref, out_ref,
             S, LB, RB, nbuf, OB,
             rsa_s, rsa_r, rsb_s, rsb_r, aga_s, aga_r, agb_s, agb_r):
        d = lax.axis_index("i")

        def lut(idx, table):
            v = jnp.int32(table[-1])
            for k in range(len(table) - 2, -1, -1):
                v = jnp.where(idx == k, jnp.int32(table[k]), v)
            return v

        p = lut(d, POS)
        rd = lut(d, RIGHT)
        ld = lut(d, LEFT)

        barrier = pltpu.get_barrier_semaphore()
        for nbr in (ld, rd):
            pl.semaphore_signal(barrier, inc=1, device_id=(nbr,),
                                device_id_type=pl.DeviceIdType.MESH)
        pl.semaphore_wait(barrier, 2)

        def compute_batch(b):
            qb = lax.dot_general(x_ref[b], wq_ref[...],
                                 (((1,), (0,)), ((), ())),
                                 preferred_element_type=jnp.float32)
            for g in range(Hkv):
                kg = k_ref[b, :, g, :]
                vg = v_ref[b, :, g, :]
                for hh in range(Hq // Hkv):
                    h = g * (Hq // Hkv) + hh
                    qh = qb[:, h * Dh:(h + 1) * Dh]
                    s = lax.dot_general(qh, kg, (((1,), (1,)), ((), ())),
                                        preferred_element_type=jnp.float32)
                    p_ = jnp.exp(s * SCALE)
                    lh = jnp.sum(p_, axis=1, keepdims=True)
                    a = lax.dot_general(p_, vg, (((1,), (0,)), ((), ())),
                                        preferred_element_type=jnp.float32)
                    S[pl.ds(b * Sq, Sq), h * Dh:(h + 1) * Dh] = a
                    S[pl.ds(b * Sq, Sq), D + h:D + h + 1] = lh

        bp = lax.div(p, jnp.int32(B))
        compute_batch(lax.rem(bp + 2, jnp.int32(B)))
        compute_batch(lax.rem(bp + 3, jnp.int32(B)))

        HC = CH // 2

        def rs_copy(stream, t, half):
            if stream == "lw":
                cs = lax.rem(p - 7 + t + N_DEV, N_DEV)
                buf, ssem, rsem, dev = LB, rsa_s, rsa_r, ld
            else:
                cs = lax.rem(p + 8 - t, N_DEV)
                buf, ssem, rsem, dev = RB, rsb_s, rsb_r, rd
            return pltpu.make_async_remote_copy(
                src_ref=S.at[pl.ds(cs * CH + half * HC, HC)],
                dst_ref=buf.at[t, pl.ds(half * HC, HC)],
                send_sem=ssem.at[2 * t + half],
                recv_sem=rsem.at[2 * t + half],
                device_id=(dev,), device_id_type=pl.DeviceIdType.MESH)

        def rs_acc(stream, t, half):
            if stream == "lw":
                c = lax.rem(p - 6 + t + N_DEV, N_DEV)
                buf = LB
            else:
                c = lax.rem(p + 7 - t, N_DEV)
                buf = RB
            r = c * CH + half * HC
            S[pl.ds(r, HC), :] = (
                S[pl.ds(r, HC), :] + buf[t, half * HC:(half + 1) * HC, :])

        rs_d = {}

        def rs_step(t):
            for half in (0, 1):
                for stream, steps in (("lw", 8), ("rw", 7)):
                    if t >= steps:
                        continue
                    rs_d[(stream, t, half)].wait()
                    rs_acc(stream, t, half)
                    if t + 1 < steps:
                        nxt = rs_copy(stream, t + 1, half)
                        rs_d[(stream, t + 1, half)] = nxt
                        nxt.start()

        for half in (0, 1):
            for stream in ("lw", "rw"):
                rs_d[(stream, 0, half)] = rs_copy(stream, 0, half)
                rs_d[(stream, 0, half)].start()
        compute_batch(lax.rem(bp + 1, jnp.int32(B)))
        rs_step(0)
        compute_batch(bp)
        for t in range(1, 8):
            rs_step(t)

        r0 = lax.rem(p + 1, N_DEV) * CH
        for h in range(Hq):
            nbuf[:, h * Dh:(h + 1) * Dh] = (
                S[pl.ds(r0, CH), h * Dh:(h + 1) * Dh]
                / S[pl.ds(r0, CH), D + h:D + h + 1])
        o = lax.dot_general(nbuf[...], wo_ref[...], (((1,), (0,)), ((), ())),
                            preferred_element_type=jnp.float32)
        out_ref[pl.ds(r0, CH), :] = o
        OB[pl.ds(r0, CH), :] = o.astype(jnp.bfloat16)

        def ag_copy(stream, t, half):
            if stream == "ar":
                cs = lax.rem(p + 1 - t + N_DEV, N_DEV)
                ssem, rsem, dev = aga_s, aga_r, rd
            else:
                cs = lax.rem(p + 1 + t, N_DEV)
                ssem, rsem, dev = agb_s, agb_r, ld
            sl = OB.at[pl.ds(cs * CH + half * HC, HC)]
            return pltpu.make_async_remote_copy(
                src_ref=sl, dst_ref=sl,
                send_sem=ssem.at[2 * t + half],
                recv_sem=rsem.at[2 * t + half],
                device_id=(dev,), device_id_type=pl.DeviceIdType.MESH)

        ag_d = {}
        for half in (0, 1):
            for stream in ("ar", "al"):
                ag_d[(stream, 0, half)] = ag_copy(stream, 0, half)
                ag_d[(stream, 0, half)].start()
        for t in range(8):
            for half in (0, 1):
                for stream, steps in (("ar", 8), ("al", 7)):
                    if t >= steps:
                        continue
                    ag_d[(stream, t, half)].wait()
                    if t + 1 < steps:
                        nxt = ag_copy(stream, t + 1, half)
                        ag_d[(stream, t + 1, half)] = nxt
                        nxt.start()
                    cr = lax.rem(
                        (p - t if stream == "ar" else p + 2 + t) + N_DEV,
                        N_DEV)
                    rr = cr * CH + half * HC
                    out_ref[pl.ds(rr, HC), :] = (
                        OB[pl.ds(rr, HC), :].astype(jnp.float32))

    flat = pl.pallas_call(
        body,
        out_shape=jax.ShapeDtypeStruct((ROWS, D), jnp.float32),
        in_specs=[pl.BlockSpec(memory_space=pltpu.VMEM)] * 5,
        out_specs=pl.BlockSpec(memory_space=pltpu.VMEM),
        scratch_shapes=[
            pltpu.VMEM((ROWS, SW), jnp.float32),
            pltpu.VMEM((8, CH, SW), jnp.float32),
            pltpu.VMEM((7, CH, SW), jnp.float32),
            pltpu.VMEM((CH, D), jnp.float32),
            pltpu.VMEM((ROWS, D), jnp.bfloat16),
            pltpu.SemaphoreType.DMA((16,)),
            pltpu.SemaphoreType.DMA((16,)),
            pltpu.SemaphoreType.DMA((14,)),
            pltpu.SemaphoreType.DMA((14,)),
            pltpu.SemaphoreType.DMA((16,)),
            pltpu.SemaphoreType.DMA((16,)),
            pltpu.SemaphoreType.DMA((14,)),
            pltpu.SemaphoreType.DMA((14,)),
        ],
        compiler_params=pltpu.CompilerParams(collective_id=0),
    )(x, Wq, Wo, K_ext, V_ext)
    return flat.reshape(B, Sq, D)


# device time: 70212 ns/iter; 2.5237x vs baseline; 1.0963x over previous
import jax
import jax.numpy as jnp
from jax import lax
from jax.experimental import pallas as pl
from jax.experimental.pallas import tpu as pltpu

N_DEV = 16
B, Sq, Hq, Hkv, Dh, D = 4, 256, 8, 2, 128, 1024
ROWS = B * Sq
CH = ROWS // N_DEV
SW = D + 128
SCALE = 0.08838834764831843

RING = [0, 4, 8, 12, 15, 11, 7, 3, 2, 6, 10, 14, 13, 9, 5, 1]
POS = [0] * N_DEV
for _i, _dev in enumerate(RING):
    POS[_dev] = _i
RIGHT = [0] * N_DEV
LEFT = [0] * N_DEV
for _i, _dev in enumerate(RING):
    RIGHT[_dev] = RING[(_i + 1) % N_DEV]
    LEFT[_dev] = RING[(_i - 1) % N_DEV]


def kernel(x, Wq, Wo, K_ext, V_ext):
    def body(x_ref, wq_ref, wo_ref, k_ref, v_ref, out_ref,
             S, SB, LB, RB, nbuf, OB,
             rsa_s, rsa_r, rsb_s, rsb_r, aga_s, aga_r, agb_s, agb_r):
        d = lax.axis_index("i")

        def lut(idx, table):
            v = jnp.int32(table[-1])
            for k in range(len(table) - 2, -1, -1):
                v = jnp.where(idx == k, jnp.int32(table[k]), v)
            return v

        p = lut(d, POS)
        rd = lut(d, RIGHT)
        ld = lut(d, LEFT)

        barrier = pltpu.get_barrier_semaphore()
        for nbr in (ld, rd):
            pl.semaphore_signal(barrier, inc=1, device_id=(nbr,),
                                device_id_type=pl.DeviceIdType.MESH)
        pl.semaphore_wait(barrier, 2)

        def compute_batch(b):
            qb = lax.dot_general(x_ref[b], wq_ref[...],
                                 (((1,), (0,)), ((), ())),
                                 preferred_element_type=jnp.float32)
            for g in range(Hkv):
                kg = k_ref[b, :, g, :]
                vg = v_ref[b, :, g, :]
                for hh in range(Hq // Hkv):
                    h = g * (Hq // Hkv) + hh
                    qh = qb[:, h * Dh:(h + 1) * Dh]
                    s = lax.dot_general(qh, kg, (((1,), (1,)), ((), ())),
                                        preferred_element_type=jnp.float32)
                    p_ = jnp.exp(s * SCALE)
                    lh = jnp.sum(p_, axis=1, keepdims=True)
                    a = lax.dot_general(p_, vg, (((1,), (0,)), ((), ())),
                                        preferred_element_type=jnp.float32)
                    S[pl.ds(b * Sq, Sq), h * Dh:(h + 1) * Dh] = a
                    S[pl.ds(b * Sq, Sq), D + h:D + h + 1] = lh
            SB[pl.ds(b * Sq, Sq), :] = (
                S[pl.ds(b * Sq, Sq), :].astype(jnp.bfloat16))

        bp = lax.div(p, jnp.int32(B))
        compute_batch(lax.rem(bp + 2, jnp.int32(B)))
        compute_batch(lax.rem(bp + 3, jnp.int32(B)))

        HC = CH // 2

        def rs_copy(stream, t, half):
            if stream == "lw":
                cs = lax.rem(p - 7 + t + N_DEV, N_DEV)
                buf, ssem, rsem, dev = LB, rsa_s, rsa_r, ld
            else:
                cs = lax.rem(p + 8 - t, N_DEV)
                buf, ssem, rsem, dev = RB, rsb_s, rsb_r, rd
            return pltpu.make_async_remote_copy(
                src_ref=SB.at[pl.ds(cs * CH + half * HC, HC)],
                dst_ref=buf.at[t, pl.ds(half * HC, HC)],
                send_sem=ssem.at[2 * t + half],
                recv_sem=rsem.at[2 * t + half],
                device_id=(dev,), device_id_type=pl.DeviceIdType.MESH)

        def rs_acc(stream, t, half, forward):
            if stream == "lw":
                c = lax.rem(p - 6 + t + N_DEV, N_DEV)
                buf = LB
            else:
                c = lax.rem(p + 7 - t, N_DEV)
                buf = RB
            r = c * CH + half * HC
            S[pl.ds(r, HC), :] = (
                S[pl.ds(r, HC), :]
                + buf[t, half * HC:(half + 1) * HC, :].astype(jnp.float32))
            if forward:
                SB[pl.ds(r, HC), :] = S[pl.ds(r, HC), :].astype(jnp.bfloat16)

        rs_d = {}

        def rs_step(t):
            for half in (0, 1):
                for stream, steps in (("lw", 8), ("rw", 7)):
                    if t >= steps:
                        continue
                    rs_d[(stream, t, half)].wait()
                    rs_acc(stream, t, half, forward=t + 1 < steps)
                    if t + 1 < steps:
                        nxt = rs_copy(stream, t + 1, half)
                        rs_d[(stream, t + 1, half)] = nxt
                        nxt.start()

        for half in (0, 1):
            for stream in ("lw", "rw"):
                rs_d[(stream, 0, half)] = rs_copy(stream, 0, half)
                rs_d[(stream, 0, half)].start()
        compute_batch(lax.rem(bp + 1, jnp.int32(B)))
        rs_step(0)
        compute_batch(bp)
        for t in range(1, 8):
            rs_step(t)

        r0 = lax.rem(p + 1, N_DEV) * CH
        for h in range(Hq):
            nbuf[:, h * Dh:(h + 1) * Dh] = (
                S[pl.ds(r0, CH), h * Dh:(h + 1) * Dh]
                / S[pl.ds(r0, CH), D + h:D + h + 1])
        o = lax.dot_general(nbuf[...], wo_ref[...], (((1,), (0,)), ((), ())),
                            preferred_element_type=jnp.float32)
        out_ref[pl.ds(r0, CH), :] = o
        OB[pl.ds(r0, CH), :] = o.astype(jnp.bfloat16)

        def ag_copy(stream, t, half):
            if stream == "ar":
                cs = lax.rem(p + 1 - t + N_DEV, N_DEV)
                ssem, rsem, dev = aga_s, aga_r, rd
            else:
                cs = lax.rem(p + 1 + t, N_DEV)
                ssem, rsem, dev = agb_s, agb_r, ld
            sl = OB.at[pl.ds(cs * CH + half * HC, HC)]
            return pltpu.make_async_remote_copy(
                src_ref=sl, dst_ref=sl,
                send_sem=ssem.at[2 * t + half],
                recv_sem=rsem.at[2 * t + half],
                device_id=(dev,), device_id_type=pl.DeviceIdType.MESH)

        ag_d = {}
        for half in (0, 1):
            for stream in ("ar", "al"):
                ag_d[(stream, 0, half)] = ag_copy(stream, 0, half)
                ag_d[(stream, 0, half)].start()
        for t in range(8):
            for half in (0, 1):
                for stream, steps in (("ar", 8), ("al", 7)):
                    if t >= steps:
                        continue
                    ag_d[(stream, t, half)].wait()
                    if t + 1 < steps:
                        nxt = ag_copy(stream, t + 1, half)
                        ag_d[(stream, t + 1, half)] = nxt
                        nxt.start()
                    cr = lax.rem(
                        (p - t if stream == "ar" else p + 2 + t) + N_DEV,
                        N_DEV)
                    rr = cr * CH + half * HC
                    out_ref[pl.ds(rr, HC), :] = (
                        OB[pl.ds(rr, HC), :].astype(jnp.float32))

    flat = pl.pallas_call(
        body,
        out_shape=jax.ShapeDtypeStruct((ROWS, D), jnp.float32),
        in_specs=[pl.BlockSpec(memory_space=pltpu.VMEM)] * 5,
        out_specs=pl.BlockSpec(memory_space=pltpu.VMEM),
        scratch_shapes=[
            pltpu.VMEM((ROWS, SW), jnp.float32),
            pltpu.VMEM((ROWS, SW), jnp.bfloat16),
            pltpu.VMEM((8, CH, SW), jnp.bfloat16),
            pltpu.VMEM((7, CH, SW), jnp.bfloat16),
            pltpu.VMEM((CH, D), jnp.float32),
            pltpu.VMEM((ROWS, D), jnp.bfloat16),
            pltpu.SemaphoreType.DMA((16,)),
            pltpu.SemaphoreType.DMA((16,)),
            pltpu.SemaphoreType.DMA((14,)),
            pltpu.SemaphoreType.DMA((14,)),
            pltpu.SemaphoreType.DMA((16,)),
            pltpu.SemaphoreType.DMA((16,)),
            pltpu.SemaphoreType.DMA((14,)),
            pltpu.SemaphoreType.DMA((14,)),
        ],
        compiler_params=pltpu.CompilerParams(collective_id=0),
    )(x, Wq, Wo, K_ext, V_ext)
    return flat.reshape(B, Sq, D)


# device time: 69263 ns/iter; 2.5583x vs baseline; 1.0137x over previous
import jax
import jax.numpy as jnp
from jax import lax
from jax.experimental import pallas as pl
from jax.experimental.pallas import tpu as pltpu

N_DEV = 16
B, Sq, Hq, Hkv, Dh, D = 4, 256, 8, 2, 128, 1024
ROWS = B * Sq
CH = ROWS // N_DEV
SW = D + 128
SCALE = 0.08838834764831843

RING = [0, 4, 8, 12, 15, 11, 7, 3, 2, 6, 10, 14, 13, 9, 5, 1]
POS = [0] * N_DEV
for _i, _dev in enumerate(RING):
    POS[_dev] = _i
RIGHT = [0] * N_DEV
LEFT = [0] * N_DEV
for _i, _dev in enumerate(RING):
    RIGHT[_dev] = RING[(_i + 1) % N_DEV]
    LEFT[_dev] = RING[(_i - 1) % N_DEV]


def kernel(x, Wq, Wo, K_ext, V_ext):
    def body(x_ref, wq_ref, wo_ref, k_ref, v_ref, out_ref,
             S, SB, LB, RB, nbuf, OB,
             rsa_s, rsa_r, rsb_s, rsb_r, aga_s, aga_r, agb_s, agb_r):
        d = lax.axis_index("i")

        def lut(idx, table):
            v = jnp.int32(table[-1])
            for k in range(len(table) - 2, -1, -1):
                v = jnp.where(idx == k, jnp.int32(table[k]), v)
            return v

        p = lut(d, POS)
        rd = lut(d, RIGHT)
        ld = lut(d, LEFT)

        barrier = pltpu.get_barrier_semaphore()
        for nbr in (ld, rd):
            pl.semaphore_signal(barrier, inc=1, device_id=(nbr,),
                                device_id_type=pl.DeviceIdType.MESH)
        pl.semaphore_wait(barrier, 2)

        def compute_batch(b):
            qb = lax.dot_general(x_ref[b], wq_ref[...],
                                 (((1,), (0,)), ((), ())),
                                 preferred_element_type=jnp.float32)
            for g in range(Hkv):
                kg = k_ref[b, :, g, :]
                vg = v_ref[b, :, g, :]
                for hh in range(Hq // Hkv):
                    h = g * (Hq // Hkv) + hh
                    qh = qb[:, h * Dh:(h + 1) * Dh]
                    s = lax.dot_general(qh, kg, (((1,), (1,)), ((), ())),
                                        preferred_element_type=jnp.float32)
                    p_ = jnp.exp(s * SCALE)
                    lh = jnp.sum(p_, axis=1, keepdims=True)
                    a = lax.dot_general(p_, vg, (((1,), (0,)), ((), ())),
                                        preferred_element_type=jnp.float32)
                    S[pl.ds(b * Sq, Sq), h * Dh:(h + 1) * Dh] = a
                    S[pl.ds(b * Sq, Sq), D + h:D + h + 1] = lh
            SB[pl.ds(b * Sq, Sq), :] = (
                S[pl.ds(b * Sq, Sq), :].astype(jnp.bfloat16))

        bp = lax.div(p, jnp.int32(B))
        compute_batch(lax.rem(bp + 2, jnp.int32(B)))
        compute_batch(lax.rem(bp + 3, jnp.int32(B)))

        HC = CH // 2
        RSUB = 4
        RHC = CH // RSUB

        def rs_copy(stream, t, sub):
            if stream == "lw":
                cs = lax.rem(p - 7 + t + N_DEV, N_DEV)
                buf, ssem, rsem, dev = LB, rsa_s, rsa_r, ld
            else:
                cs = lax.rem(p + 8 - t, N_DEV)
                buf, ssem, rsem, dev = RB, rsb_s, rsb_r, rd
            return pltpu.make_async_remote_copy(
                src_ref=SB.at[pl.ds(cs * CH + sub * RHC, RHC)],
                dst_ref=buf.at[t, pl.ds(sub * RHC, RHC)],
                send_sem=ssem.at[RSUB * t + sub],
                recv_sem=rsem.at[RSUB * t + sub],
                device_id=(dev,), device_id_type=pl.DeviceIdType.MESH)

        def rs_acc(stream, t, sub, forward):
            if stream == "lw":
                c = lax.rem(p - 6 + t + N_DEV, N_DEV)
                buf = LB
            else:
                c = lax.rem(p + 7 - t, N_DEV)
                buf = RB
            r = c * CH + sub * RHC
            S[pl.ds(r, RHC), :] = (
                S[pl.ds(r, RHC), :]
                + buf[t, sub * RHC:(sub + 1) * RHC, :].astype(jnp.float32))
            if forward:
                SB[pl.ds(r, RHC), :] = S[pl.ds(r, RHC), :].astype(jnp.bfloat16)

        rs_d = {}

        def rs_step(t):
            for sub in range(RSUB):
                for stream, steps in (("lw", 8), ("rw", 7)):
                    if t >= steps:
                        continue
                    rs_d[(stream, t, sub)].wait()
                    rs_acc(stream, t, sub, forward=t + 1 < steps)
                    if t + 1 < steps:
                        nxt = rs_copy(stream, t + 1, sub)
                        rs_d[(stream, t + 1, sub)] = nxt
                        nxt.start()

        for sub in range(RSUB):
            for stream in ("lw", "rw"):
                rs_d[(stream, 0, sub)] = rs_copy(stream, 0, sub)
                rs_d[(stream, 0, sub)].start()
        compute_batch(lax.rem(bp + 1, jnp.int32(B)))
        rs_step(0)
        compute_batch(bp)
        for t in range(1, 8):
            rs_step(t)

        r0 = lax.rem(p + 1, N_DEV) * CH
        for h in range(Hq):
            nbuf[:, h * Dh:(h + 1) * Dh] = (
                S[pl.ds(r0, CH), h * Dh:(h + 1) * Dh]
                / S[pl.ds(r0, CH), D + h:D + h + 1])
        o = lax.dot_general(nbuf[...], wo_ref[...], (((1,), (0,)), ((), ())),
                            preferred_element_type=jnp.float32)
        out_ref[pl.ds(r0, CH), :] = o
        OB[pl.ds(r0, CH), :] = o.astype(jnp.bfloat16)

        def ag_copy(stream, t, half):
            if stream == "ar":
                cs = lax.rem(p + 1 - t + N_DEV, N_DEV)
                ssem, rsem, dev = aga_s, aga_r, rd
            else:
                cs = lax.rem(p + 1 + t, N_DEV)
                ssem, rsem, dev = agb_s, agb_r, ld
            sl = OB.at[pl.ds(cs * CH + half * HC, HC)]
            return pltpu.make_async_remote_copy(
                src_ref=sl, dst_ref=sl,
                send_sem=ssem.at[2 * t + half],
                recv_sem=rsem.at[2 * t + half],
                device_id=(dev,), device_id_type=pl.DeviceIdType.MESH)

        ag_d = {}
        for half in (0, 1):
            for stream in ("ar", "al"):
                ag_d[(stream, 0, half)] = ag_copy(stream, 0, half)
                ag_d[(stream, 0, half)].start()
        for t in range(8):
            for half in (0, 1):
                for stream, steps in (("ar", 8), ("al", 7)):
                    if t >= steps:
                        continue
                    ag_d[(stream, t, half)].wait()
                    if t + 1 < steps:
                        nxt = ag_copy(stream, t + 1, half)
                        ag_d[(stream, t + 1, half)] = nxt
                        nxt.start()
                    cr = lax.rem(
                        (p - t if stream == "ar" else p + 2 + t) + N_DEV,
                        N_DEV)
                    rr = cr * CH + half * HC
                    out_ref[pl.ds(rr, HC), :] = (
                        OB[pl.ds(rr, HC), :].astype(jnp.float32))

    flat = pl.pallas_call(
        body,
        out_shape=jax.ShapeDtypeStruct((ROWS, D), jnp.float32),
        in_specs=[pl.BlockSpec(memory_space=pltpu.VMEM)] * 5,
        out_specs=pl.BlockSpec(memory_space=pltpu.VMEM),
        scratch_shapes=[
            pltpu.VMEM((ROWS, SW), jnp.float32),
            pltpu.VMEM((ROWS, SW), jnp.bfloat16),
            pltpu.VMEM((8, CH, SW), jnp.bfloat16),
            pltpu.VMEM((7, CH, SW), jnp.bfloat16),
            pltpu.VMEM((CH, D), jnp.float32),
            pltpu.VMEM((ROWS, D), jnp.bfloat16),
            pltpu.SemaphoreType.DMA((32,)),
            pltpu.SemaphoreType.DMA((32,)),
            pltpu.SemaphoreType.DMA((28,)),
            pltpu.SemaphoreType.DMA((28,)),
            pltpu.SemaphoreType.DMA((16,)),
            pltpu.SemaphoreType.DMA((16,)),
            pltpu.SemaphoreType.DMA((14,)),
            pltpu.SemaphoreType.DMA((14,)),
        ],
        compiler_params=pltpu.CompilerParams(collective_id=0),
    )(x, Wq, Wo, K_ext, V_ext)
    return flat.reshape(B, Sq, D)


# device time: 68155 ns/iter; 2.5999x vs baseline; 1.0163x over previous
import jax
import jax.numpy as jnp
from jax import lax
from jax.experimental import pallas as pl
from jax.experimental.pallas import tpu as pltpu

N_DEV = 16
B, Sq, Hq, Hkv, Dh, D = 4, 256, 8, 2, 128, 1024
ROWS = B * Sq
CH = ROWS // N_DEV
SW = D + 128
SCALE = 0.08838834764831843

RING = [0, 4, 8, 12, 15, 11, 7, 3, 2, 6, 10, 14, 13, 9, 5, 1]
POS = [0] * N_DEV
for _i, _dev in enumerate(RING):
    POS[_dev] = _i
RIGHT = [0] * N_DEV
LEFT = [0] * N_DEV
for _i, _dev in enumerate(RING):
    RIGHT[_dev] = RING[(_i + 1) % N_DEV]
    LEFT[_dev] = RING[(_i - 1) % N_DEV]


def kernel(x, Wq, Wo, K_ext, V_ext):
    def body(x_ref, wq_ref, wo_ref, k_ref, v_ref, out_ref,
             S, SB, LB, RB, nbuf, OB,
             rsa_s, rsa_r, rsb_s, rsb_r, aga_s, aga_r, agb_s, agb_r):
        d = lax.axis_index("i")

        def lut(idx, table):
            v = jnp.int32(table[-1])
            for k in range(len(table) - 2, -1, -1):
                v = jnp.where(idx == k, jnp.int32(table[k]), v)
            return v

        p = lut(d, POS)
        rd = lut(d, RIGHT)
        ld = lut(d, LEFT)

        barrier = pltpu.get_barrier_semaphore()
        for nbr in (ld, rd):
            pl.semaphore_signal(barrier, inc=1, device_id=(nbr,),
                                device_id_type=pl.DeviceIdType.MESH)
        pl.semaphore_wait(barrier, 2)

        def compute_batch(b):
            qb = lax.dot_general(x_ref[b], wq_ref[...],
                                 (((1,), (0,)), ((), ())),
                                 preferred_element_type=jnp.float32)
            for g in range(Hkv):
                kg = k_ref[b, :, g, :]
                vg = v_ref[b, :, g, :]
                for hh in range(Hq // Hkv):
                    h = g * (Hq // Hkv) + hh
                    qh = qb[:, h * Dh:(h + 1) * Dh]
                    s = lax.dot_general(qh, kg, (((1,), (1,)), ((), ())),
                                        preferred_element_type=jnp.float32)
                    p_ = jnp.exp(s * SCALE)
                    lh = jnp.sum(p_, axis=1, keepdims=True)
                    a = lax.dot_general(p_, vg, (((1,), (0,)), ((), ())),
                                        preferred_element_type=jnp.float32)
                    S[pl.ds(b * Sq, Sq), h * Dh:(h + 1) * Dh] = a
                    S[pl.ds(b * Sq, Sq), D + h:D + h + 1] = lh
            SB[pl.ds(b * Sq, Sq), :] = (
                S[pl.ds(b * Sq, Sq), :].astype(jnp.bfloat16))

        bp = lax.div(p, jnp.int32(B))
        compute_batch(lax.rem(bp + 2, jnp.int32(B)))
        compute_batch(lax.rem(bp + 3, jnp.int32(B)))

        HC = CH // 2
        RSUB = 4
        RHC = CH // RSUB

        def rs_copy(stream, t, sub):
            if stream == "lw":
                cs = lax.rem(p - 7 + t + N_DEV, N_DEV)
                buf, ssem, rsem, dev = LB, rsa_s, rsa_r, ld
            else:
                cs = lax.rem(p + 8 - t, N_DEV)
                buf, ssem, rsem, dev = RB, rsb_s, rsb_r, rd
            return pltpu.make_async_remote_copy(
                src_ref=SB.at[pl.ds(cs * CH + sub * RHC, RHC)],
                dst_ref=buf.at[t, pl.ds(sub * RHC, RHC)],
                send_sem=ssem.at[RSUB * t + sub],
                recv_sem=rsem.at[RSUB * t + sub],
                device_id=(dev,), device_id_type=pl.DeviceIdType.MESH)

        def rs_acc(stream, t, sub, forward):
            if stream == "lw":
                c = lax.rem(p - 6 + t + N_DEV, N_DEV)
                buf = LB
            else:
                c = lax.rem(p + 7 - t, N_DEV)
                buf = RB
            r = c * CH + sub * RHC
            if forward:
                SB[pl.ds(r, RHC), :] = (
                    SB[pl.ds(r, RHC), :]
                    + buf[t, sub * RHC:(sub + 1) * RHC, :])
            else:
                S[pl.ds(r, RHC), :] = (
                    S[pl.ds(r, RHC), :]
                    + buf[t, sub * RHC:(sub + 1) * RHC, :].astype(jnp.float32))

        rs_d = {}

        def rs_step(t):
            for sub in range(RSUB):
                for stream, steps in (("lw", 8), ("rw", 7)):
                    if t >= steps:
                        continue
                    rs_d[(stream, t, sub)].wait()
                    rs_acc(stream, t, sub, forward=t + 1 < steps)
                    if t + 1 < steps:
                        nxt = rs_copy(stream, t + 1, sub)
                        rs_d[(stream, t + 1, sub)] = nxt
                        nxt.start()

        for sub in range(RSUB):
            for stream in ("lw", "rw"):
                rs_d[(stream, 0, sub)] = rs_copy(stream, 0, sub)
                rs_d[(stream, 0, sub)].start()
        compute_batch(lax.rem(bp + 1, jnp.int32(B)))
        rs_step(0)
        compute_batch(bp)
        for t in range(1, 8):
            rs_step(t)

        r0 = lax.rem(p + 1, N_DEV) * CH
        for h in range(Hq):
            nbuf[:, h * Dh:(h + 1) * Dh] = (
                S[pl.ds(r0, CH), h * Dh:(h + 1) * Dh]
                / S[pl.ds(r0, CH), D + h:D + h + 1])
        o = lax.dot_general(nbuf[...], wo_ref[...], (((1,), (0,)), ((), ())),
                            preferred_element_type=jnp.float32)
        out_ref[pl.ds(r0, CH), :] = o
        OB[pl.ds(r0, CH), :] = o.astype(jnp.bfloat16)

        def ag_copy(stream, t, half):
            if stream == "ar":
                cs = lax.rem(p + 1 - t + N_DEV, N_DEV)
                ssem, rsem, dev = aga_s, aga_r, rd
            else:
                cs = lax.rem(p + 1 + t, N_DEV)
                ssem, rsem, dev = agb_s, agb_r, ld
            sl = OB.at[pl.ds(cs * CH + half * HC, HC)]
            return pltpu.make_async_remote_copy(
                src_ref=sl, dst_ref=sl,
                send_sem=ssem.at[2 * t + half],
                recv_sem=rsem.at[2 * t + half],
                device_id=(dev,), device_id_type=pl.DeviceIdType.MESH)

        ag_d = {}
        for half in (0, 1):
            for stream in ("ar", "al"):
                ag_d[(stream, 0, half)] = ag_copy(stream, 0, half)
                ag_d[(stream, 0, half)].start()
        for t in range(8):
            for half in (0, 1):
                for stream, steps in (("ar", 8), ("al", 7)):
                    if t >= steps:
                        continue
                    ag_d[(stream, t, half)].wait()
                    if t + 1 < steps:
                        nxt = ag_copy(stream, t + 1, half)
                        ag_d[(stream, t + 1, half)] = nxt
                        nxt.start()
                    cr = lax.rem(
                        (p - t if stream == "ar" else p + 2 + t) + N_DEV,
                        N_DEV)
                    rr = cr * CH + half * HC
                    out_ref[pl.ds(rr, HC), :] = (
                        OB[pl.ds(rr, HC), :].astype(jnp.float32))

    flat = pl.pallas_call(
        body,
        out_shape=jax.ShapeDtypeStruct((ROWS, D), jnp.float32),
        in_specs=[pl.BlockSpec(memory_space=pltpu.VMEM)] * 5,
        out_specs=pl.BlockSpec(memory_space=pltpu.VMEM),
        scratch_shapes=[
            pltpu.VMEM((ROWS, SW), jnp.float32),
            pltpu.VMEM((ROWS, SW), jnp.bfloat16),
            pltpu.VMEM((8, CH, SW), jnp.bfloat16),
            pltpu.VMEM((7, CH, SW), jnp.bfloat16),
            pltpu.VMEM((CH, D), jnp.float32),
            pltpu.VMEM((ROWS, D), jnp.bfloat16),
            pltpu.SemaphoreType.DMA((32,)),
            pltpu.SemaphoreType.DMA((32,)),
            pltpu.SemaphoreType.DMA((28,)),
            pltpu.SemaphoreType.DMA((28,)),
            pltpu.SemaphoreType.DMA((16,)),
            pltpu.SemaphoreType.DMA((16,)),
            pltpu.SemaphoreType.DMA((14,)),
            pltpu.SemaphoreType.DMA((14,)),
        ],
        compiler_params=pltpu.CompilerParams(collective_id=0),
    )(x, Wq, Wo, K_ext, V_ext)
    return flat.reshape(B, Sq, D)
